# Initial kernel scaffold; baseline (speedup 1.0000x reference)
import jax
import jax.numpy as jnp
from jax.experimental import pallas as pl


def kernel(features, edge_index, W0, b0, W1, b1, W2, b2):
    raise NotImplementedError("write your pallas kernel here")


# speedup vs baseline: 4.4307x; 4.4307x over previous
import functools

import jax
import jax.numpy as jnp
from jax import lax
from jax.experimental import pallas as pl
from jax.experimental.pallas import tpu as pltpu
from jax.experimental.pallas import tpu_sc as plsc

N = 10000
NP = 10240
E = 320000
F = 64
IN_F = 128
K = 10
ALPHA = 0.1
NT = 16
NPT = NP // NT
CHUNK = 128
NCH = 158
EPT = NCH * CHUNK
EP = NT * EPT
UCH = 64
NUC = NPT // UCH
DEGW = 16

_mesh = plsc.VectorSubcoreMesh(core_axis_name="c", subcore_axis_name="s",
                               num_cores=1)
_sc_params = pltpu.CompilerParams(use_tc_tiling_on_sc=False)


def _fill2d(ref, rows, val):
    cols = ref.shape[1] // 16

    def body(r, _):
        for f in range(cols):
            ref[r, pl.ds(16 * f, 16)] = jnp.full((16,), val, jnp.float32)
        return 0

    lax.fori_loop(0, rows, body, 0)


def _deg_body(src_h, dst_h, odeg_h, ideg_h,
              osh, ish, sidx, didx, ones_v, zv):
    s = lax.axis_index("s")
    nbase = s * NPT
    pltpu.sync_copy(src_h.at[s], sidx)
    pltpu.sync_copy(dst_h.at[s], didx)
    _fill2d(ones_v, CHUNK, 1.0)
    _fill2d(zv, UCH, 0.0)
    for i in range(NPT // UCH):
        pltpu.sync_copy(zv, osh.at[pl.ds(nbase + i * UCH, UCH)])
        pltpu.sync_copy(zv, ish.at[pl.ds(nbase + i * UCH, UCH)])
    plsc.subcore_barrier()

    def chunk(j, _):
        pltpu.sync_copy(ones_v, osh.at[sidx.at[j]], add=True)
        pltpu.sync_copy(ones_v, ish.at[didx.at[j]], add=True)
        return 0

    lax.fori_loop(0, NCH, chunk, 0)
    plsc.subcore_barrier()
    pltpu.sync_copy(osh.at[pl.ds(nbase, NPT)], odeg_h.at[pl.ds(nbase, NPT)])
    pltpu.sync_copy(ish.at[pl.ds(nbase, NPT)], ideg_h.at[pl.ds(nbase, NPT)])


@jax.jit
def _deg(srcp, dstp):
    return pl.kernel(
        _deg_body,
        out_type=[jax.ShapeDtypeStruct((NP, DEGW), jnp.float32),
                  jax.ShapeDtypeStruct((NP, DEGW), jnp.float32)],
        mesh=_mesh,
        scratch_types=[
            pltpu.VMEM_SHARED((NP, DEGW), jnp.float32),
            pltpu.VMEM_SHARED((NP, DEGW), jnp.float32),
            pltpu.VMEM((NCH, CHUNK), jnp.int32),
            pltpu.VMEM((NCH, CHUNK), jnp.int32),
            pltpu.VMEM((CHUNK, DEGW), jnp.float32),
            pltpu.VMEM((UCH, DEGW), jnp.float32),
        ],
        compiler_params=_sc_params,
    )(srcp, dstp)


def _mlp_body(x_ref, od_ref, id_ref, w0_ref, b0_ref, w1_ref, b1_ref,
              w2_ref, b2_ref, g0_ref, c1_ref, d2_ref):
    x = x_ref[...]
    h = jnp.maximum(jnp.dot(x, w0_ref[...],
                            preferred_element_type=jnp.float32) + b0_ref[...], 0.0)
    h = jnp.maximum(jnp.dot(h, w1_ref[...],
                            preferred_element_type=jnp.float32) + b1_ref[...], 0.0)
    h0 = jnp.dot(h, w2_ref[...], preferred_element_type=jnp.float32) + b2_ref[...]
    od = jnp.maximum(od_ref[...], 1.0)
    idg = jnp.maximum(id_ref[...], 1.0)
    onrm = lax.rsqrt(od)
    inrm = lax.rsqrt(idg)
    g0_ref[...] = h0 * onrm
    c1_ref[...] = (1.0 - ALPHA) * onrm * inrm
    d2_ref[...] = jnp.sqrt(od)


@jax.jit
def _mlp(xp, od, idg, w0, b0, w1, b1, w2, b2):
    blk = NPT
    grid = (NP // blk,)
    return pl.pallas_call(
        _mlp_body,
        grid=grid,
        in_specs=[
            pl.BlockSpec((blk, IN_F), lambda i: (i, 0)),
            pl.BlockSpec((blk, 1), lambda i: (i, 0)),
            pl.BlockSpec((blk, 1), lambda i: (i, 0)),
            pl.BlockSpec((IN_F, IN_F), lambda i: (0, 0)),
            pl.BlockSpec((1, IN_F), lambda i: (0, 0)),
            pl.BlockSpec((IN_F, IN_F), lambda i: (0, 0)),
            pl.BlockSpec((1, IN_F), lambda i: (0, 0)),
            pl.BlockSpec((IN_F, F), lambda i: (0, 0)),
            pl.BlockSpec((1, F), lambda i: (0, 0)),
        ],
        out_specs=[
            pl.BlockSpec((blk, F), lambda i: (i, 0)),
            pl.BlockSpec((blk, 1), lambda i: (i, 0)),
            pl.BlockSpec((blk, 1), lambda i: (i, 0)),
        ],
        out_shape=[jax.ShapeDtypeStruct((NP, F), jnp.float32),
                   jax.ShapeDtypeStruct((NP, 1), jnp.float32),
                   jax.ShapeDtypeStruct((NP, 1), jnp.float32)],
    )(xp, od, idg, w0, b0, w1, b1, w2, b2)


def _prop_body(src_h, dst_h, g0_h, c1_h, d2_h, h_out, gbuf_h,
               agg_sh, sidx, didx, rows0, g0ch, c1v, d2v, aggv, gnew, zv):
    s = lax.axis_index("s")
    nbase = s * NPT
    pltpu.sync_copy(src_h.at[s], sidx)
    pltpu.sync_copy(dst_h.at[s], didx)
    pltpu.sync_copy(c1_h.at[pl.ds(nbase, NPT)], c1v.at[pl.ds(0, NPT)])
    pltpu.sync_copy(d2_h.at[pl.ds(nbase, NPT)], d2v.at[pl.ds(0, NPT)])
    _fill2d(zv, UCH, 0.0)
    for i in range(NUC):
        pltpu.sync_copy(zv, agg_sh.at[pl.ds(nbase + i * UCH, UCH)])
    plsc.subcore_barrier()

    def _edge_phase(table_h):
        def chunk(j, _c):
            pltpu.sync_copy(table_h.at[sidx.at[j]], rows0)
            pltpu.sync_copy(rows0, agg_sh.at[didx.at[j]], add=True)
            return 0

        lax.fori_loop(0, NCH, chunk, 0)

    def step(t, _):
        @pl.when(t == 0)
        def _():
            _edge_phase(g0_h)

        @pl.when(t > 0)
        def _():
            _edge_phase(gbuf_h)

        plsc.subcore_barrier()

        for i in range(NUC):
            nb_l = i * UCH
            nb = nbase + nb_l
            pltpu.sync_copy(agg_sh.at[pl.ds(nb, UCH)], aggv)
            pltpu.sync_copy(zv, agg_sh.at[pl.ds(nb, UCH)])
            pltpu.sync_copy(g0_h.at[pl.ds(nb, UCH)], g0ch)

            def noderow(r, _n):
                c1s = c1v[pl.ds(nb_l + r, 16)][0]
                for f in range(F // 16):
                    col = pl.ds(16 * f, 16)
                    gnew[r, col] = aggv[r, col] * c1s + ALPHA * g0ch[r, col]
                return 0

            lax.fori_loop(0, UCH, noderow, 0)

            @pl.when(t < K - 1)
            def _():
                pltpu.sync_copy(gnew, gbuf_h.at[pl.ds(nb, UCH)])

            @pl.when(t == K - 1)
            def _():
                def finrow(r, _n):
                    d2s = d2v[pl.ds(nb_l + r, 16)][0]
                    for f in range(F // 16):
                        col = pl.ds(16 * f, 16)
                        gnew[r, col] = gnew[r, col] * d2s
                    return 0

                lax.fori_loop(0, UCH, finrow, 0)

                @pl.when(nb + UCH <= N)
                def _():
                    pltpu.sync_copy(gnew, h_out.at[pl.ds(nb, UCH)])

                @pl.when(jnp.logical_and(nb < N, nb + UCH > N))
                def _():
                    pltpu.sync_copy(gnew.at[pl.ds(0, N % UCH)],
                                    h_out.at[pl.ds(nb, N % UCH)])
        plsc.subcore_barrier()
        return 0

    lax.fori_loop(0, K, step, 0)


@jax.jit
def _prop(srcp, dstp, g0, c1, d2):
    return pl.kernel(
        _prop_body,
        out_type=[jax.ShapeDtypeStruct((N, F), jnp.float32),
                  jax.ShapeDtypeStruct((NP, F), jnp.float32)],
        mesh=_mesh,
        scratch_types=[
            pltpu.VMEM_SHARED((NP, F), jnp.float32),
            pltpu.VMEM((NCH, CHUNK), jnp.int32),
            pltpu.VMEM((NCH, CHUNK), jnp.int32),
            pltpu.VMEM((CHUNK, F), jnp.float32),
            pltpu.VMEM((UCH, F), jnp.float32),
            pltpu.VMEM((NPT + 16,), jnp.float32),
            pltpu.VMEM((NPT + 16,), jnp.float32),
            pltpu.VMEM((UCH, F), jnp.float32),
            pltpu.VMEM((UCH, F), jnp.float32),
            pltpu.VMEM((UCH, F), jnp.float32),
        ],
        compiler_params=_sc_params,
    )(srcp, dstp, g0, c1, d2)


def kernel(features, edge_index, W0, b0, W1, b1, W2, b2):
    src = edge_index[0].astype(jnp.int32)
    dst = edge_index[1].astype(jnp.int32)
    pad_e = EP - E
    pad_idx = jnp.full((pad_e,), NP - 1, jnp.int32)
    srcp = jnp.concatenate([src, pad_idx]).reshape(NT, NCH, CHUNK)
    dstp = jnp.concatenate([dst, pad_idx]).reshape(NT, NCH, CHUNK)
    od, idg = _deg(srcp, dstp)
    xp = jnp.concatenate(
        [features, jnp.zeros((NP - N, IN_F), jnp.float32)], axis=0)
    g0, c1, d2 = _mlp(xp, od[:, :1], idg[:, :1],
                      W0, b0.reshape(1, IN_F), W1, b1.reshape(1, IN_F),
                      W2, b2.reshape(1, F))
    h, _ = _prop(srcp, dstp, g0, c1.reshape(NP), d2.reshape(NP))
    return h
